# Initial kernel scaffold; baseline (speedup 1.0000x reference)
import jax
import jax.numpy as jnp
from jax.experimental import pallas as pl


def kernel(q, idx_t, w_t, idx_v, w_v, idx_e, w_e, time_codes, var_codes, event_codes, contribute_mask, params):
    raise NotImplementedError("write your pallas kernel here")


# speedup vs baseline: 19.2361x; 19.2361x over previous
import functools
import jax
import jax.numpy as jnp
from jax import lax
from jax.experimental import pallas as pl
from jax.experimental.pallas import tpu as pltpu

B, N, M, D = 8, 2048, 8, 128
KT, KV, KE = 64, 128, 256
K_ALL = KT + KV + KE
TM = 128
NT = N // TM


def _assemble_qlin(p):
    r, i, j, k = p['r'], p['i'], p['j'], p['k']
    W = jnp.concatenate([
        jnp.concatenate([r, -i, -j, -k], axis=1),
        jnp.concatenate([i, r, -k, j], axis=1),
        jnp.concatenate([j, k, r, -i], axis=1),
        jnp.concatenate([k, -j, i, r], axis=1)], axis=0)
    return W.T, p['b']


def _hamilton(p, x):
    Qd = p.shape[-1] // 4
    pr, pi, pj, pk = (p[..., c*Qd:(c+1)*Qd] for c in range(4))
    xr, xi, xj, xk = (x[..., c*Qd:(c+1)*Qd] for c in range(4))
    return jnp.concatenate([
        pr*xr - pi*xi - pj*xj - pk*xk,
        pr*xi + pi*xr + pj*xk - pk*xj,
        pr*xj - pi*xk + pj*xr + pk*xi,
        pr*xk + pi*xj - pj*xi + pk*xr], axis=-1)


def _scatter_body(idx_ref, w_ref, q_ref, proto_ref, ws_ref, wmat_ref, acc_ref, wsacc_ref):
    t = pl.program_id(1)
    idx = idx_ref[0]
    w = w_ref[0]
    q = q_ref[0]

    def build(mlo, mhi, K, off):
        acc = jnp.zeros((TM, K), jnp.float32)
        kio = lax.broadcasted_iota(jnp.int32, (TM, K), 1) + off
        for m in range(mlo, mhi):
            hit = (idx[:, m:m+1] == kio)
            acc = acc + jnp.where(hit, w[:, m:m+1], 0.0)
        return acc
    Wt = build(0, M, KT, 0)
    Wv = build(M, 2*M, KV, KT)
    We = build(2*M, 3*M, KE, KT + KV)
    Wfull = jnp.concatenate([Wt, Wv, We], axis=1)
    wmat_ref[0] = Wfull

    contrib = jnp.dot(Wfull.T, q, preferred_element_type=jnp.float32)
    wscon = jnp.sum(Wfull, axis=0)

    @pl.when(t == 0)
    def _():
        acc_ref[...] = jnp.zeros_like(acc_ref)
        wsacc_ref[...] = jnp.zeros_like(wsacc_ref)

    acc_ref[...] += contrib
    wsacc_ref[...] += jnp.broadcast_to(wscon[None, :], (8, K_ALL))

    @pl.when(t == NT - 1)
    def _():
        proto_ref[0] = acc_ref[...]
        ws_ref[0] = wsacc_ref[0:1]


def _scatter_call(idx_all, w_all, q):
    return pl.pallas_call(
        _scatter_body,
        grid=(B, NT),
        in_specs=[
            pl.BlockSpec((1, TM, 3*M), lambda b, t: (b, t, 0)),
            pl.BlockSpec((1, TM, 3*M), lambda b, t: (b, t, 0)),
            pl.BlockSpec((1, TM, D), lambda b, t: (b, t, 0)),
        ],
        out_specs=[
            pl.BlockSpec((1, K_ALL, D), lambda b, t: (b, 0, 0)),
            pl.BlockSpec((1, 1, K_ALL), lambda b, t: (b, 0, 0)),
            pl.BlockSpec((1, TM, K_ALL), lambda b, t: (b, t, 0)),
        ],
        out_shape=[
            jax.ShapeDtypeStruct((B, K_ALL, D), jnp.float32),
            jax.ShapeDtypeStruct((B, 1, K_ALL), jnp.float32),
            jax.ShapeDtypeStruct((B, N, K_ALL), jnp.float32),
        ],
        scratch_shapes=[
            pltpu.VMEM((K_ALL, D), jnp.float32),
            pltpu.VMEM((8, K_ALL), jnp.float32),
        ],
    )(idx_all, w_all, q)


def _mid_body(proto_ref, ws_ref, codes_ref, wq_ref, bq_ref, pb_ref):
    proto = proto_ref[0]
    ws = ws_ref[0][0]
    codes = codes_ref[...]
    wsc = jnp.maximum(ws, 0.001)[:, None]
    blend = jnp.clip(ws / (ws + 0.5), 0.0, 1.0)[:, None]
    pb = blend * (proto / wsc) + (1.0 - blend) * codes

    outs = []
    offs = [0, KT, KT + KV, K_ALL]
    for c in range(3):
        seg = pb[offs[c]:offs[c+1], :]
        wt = wq_ref[c]
        bb = bq_ref[c]
        outs.append(jnp.dot(seg, wt, preferred_element_type=jnp.float32) + bb[None, :])
    pb_ref[0] = jnp.concatenate(outs, axis=0)


def _mid_call(proto, ws, codes_all, wq_stack, bq_stack):
    return pl.pallas_call(
        _mid_body,
        grid=(B,),
        in_specs=[
            pl.BlockSpec((1, K_ALL, D), lambda b: (b, 0, 0)),
            pl.BlockSpec((1, 1, K_ALL), lambda b: (b, 0, 0)),
            pl.BlockSpec((K_ALL, D), lambda b: (0, 0)),
            pl.BlockSpec((3, D, D), lambda b: (0, 0, 0)),
            pl.BlockSpec((3, D), lambda b: (0, 0)),
        ],
        out_specs=pl.BlockSpec((1, K_ALL, D), lambda b: (b, 0, 0)),
        out_shape=jax.ShapeDtypeStruct((B, K_ALL, D), jnp.float32),
    )(proto, ws, codes_all, wq_stack, bq_stack)


def _final_body(wmat_ref, pb_ref, q_ref, wu_ref, bu_ref, g_ref, be_ref, qn_ref):
    Wfull = wmat_ref[0]
    pb = pb_ref[0]
    q = q_ref[0]
    g = jnp.dot(Wfull, pb, preferred_element_type=jnp.float32)
    h = _hamilton(q, g)
    msg = jnp.dot(h, wu_ref[...], preferred_element_type=jnp.float32) + bu_ref[...][None, :]
    x = q + msg
    Qd = D // 4
    outs = []
    for c in range(4):
        xc = x[:, c*Qd:(c+1)*Qd]
        mu = jnp.mean(xc, axis=-1, keepdims=True)
        var = jnp.mean(xc * xc, axis=-1, keepdims=True) - mu * mu
        xn = (xc - mu) * lax.rsqrt(var + 1e-5)
        outs.append(xn * g_ref[...][None, c*Qd:(c+1)*Qd] + be_ref[...][None, c*Qd:(c+1)*Qd])
    qn_ref[0] = jnp.concatenate(outs, axis=-1)


def _final_call(wmat, pb, q, wu, bu, gvec, bvec):
    return pl.pallas_call(
        _final_body,
        grid=(B, NT),
        in_specs=[
            pl.BlockSpec((1, TM, K_ALL), lambda b, t: (b, t, 0)),
            pl.BlockSpec((1, K_ALL, D), lambda b, t: (b, 0, 0)),
            pl.BlockSpec((1, TM, D), lambda b, t: (b, t, 0)),
            pl.BlockSpec((D, D), lambda b, t: (0, 0)),
            pl.BlockSpec((D,), lambda b, t: (0,)),
            pl.BlockSpec((D,), lambda b, t: (0,)),
            pl.BlockSpec((D,), lambda b, t: (0,)),
        ],
        out_specs=pl.BlockSpec((1, TM, D), lambda b, t: (b, t, 0)),
        out_shape=jax.ShapeDtypeStruct((B, N, D), jnp.float32),
    )(wmat, pb, q, wu, bu, gvec, bvec)


def kernel(q, idx_t, w_t, idx_v, w_v, idx_e, w_e, time_codes, var_codes,
           event_codes, contribute_mask, params):
    idx_all = jnp.concatenate(
        [idx_t, idx_v + KT, idx_e + (KT + KV)], axis=2).astype(jnp.int32)
    w_all = jnp.concatenate([w_t, w_v, w_e], axis=2) * contribute_mask[:, :, None]
    codes_all = jnp.concatenate([time_codes, var_codes, event_codes], axis=0)

    wq, bq = zip(*(_assemble_qlin(params[k]) for k in ('proto_t', 'proto_v', 'proto_e')))
    wq_stack = jnp.stack(wq)
    bq_stack = jnp.stack(bq)
    wu, bu = _assemble_qlin(params['update_proj'])
    gvec = jnp.concatenate(params['ln_g'])
    bvec = jnp.concatenate(params['ln_b'])

    proto, ws, wmat = _scatter_call(idx_all, w_all, q)
    pb = _mid_call(proto, ws, codes_all, wq_stack, bq_stack)
    q_new = _final_call(wmat, pb, q, wu, bu, gvec, bvec)

    proto_t = pb[:, :KT, :]
    proto_v = pb[:, KT:KT+KV, :]
    proto_e = pb[:, KT+KV:, :]
    return (q_new, proto_t, proto_v, proto_e)
